# Initial kernel scaffold; baseline (speedup 1.0000x reference)
import jax
import jax.numpy as jnp
from jax.experimental import pallas as pl


def kernel(x, edge_idx, edge_attr, Wq, bq, Wk, bk, Wv, bv, We, Ws, bs, gamma, beta):
    raise NotImplementedError("write your pallas kernel here")


# speedup vs baseline: 14.5147x; 14.5147x over previous
import math

import jax
import jax.numpy as jnp
from jax import lax
from jax.experimental import pallas as pl
from jax.experimental.pallas import tpu as pltpu
from jax.experimental.pallas import tpu_sc as plsc

N = 10000
E = 320000
F = 128
H = 4
C = 32
D = H * C

NC = 2
NS = 16
NW = NC * NS
E_PER_W = E // NW
CHUNK = 80
NCHUNK = E_PER_W // CHUNK
NPAD = 10240
ROWS_PER_TILE = NPAD // NS
SSROWS = NPAD * H // D

_RS32 = 1.0 / math.sqrt(C)

_SC_PARAMS = pltpu.CompilerParams(needs_layout_passes=False)


def _proj_body(x_ref, wq_ref, wk_ref, wv_ref, ws_ref, b_ref,
               q_ref, k_ref, v_ref, s_ref):
    xb = x_ref[...]
    b = b_ref[...]
    q_ref[...] = jnp.dot(xb, wq_ref[...], preferred_element_type=jnp.float32) + b[0:1]
    k_ref[...] = jnp.dot(xb, wk_ref[...], preferred_element_type=jnp.float32) + b[1:2]
    v_ref[...] = jnp.dot(xb, wv_ref[...], preferred_element_type=jnp.float32) + b[2:3]
    s_ref[...] = jnp.dot(xb, ws_ref[...], preferred_element_type=jnp.float32) + b[3:4]


def _projections(x, wqt, wkt, wvt, wst, bstack):
    blk = 400
    grid = N // blk
    wspec = pl.BlockSpec((F, D), lambda i: (0, 0))
    return pl.pallas_call(
        _proj_body,
        grid=(grid,),
        in_specs=[
            pl.BlockSpec((blk, F), lambda i: (i, 0)),
            wspec, wspec, wspec, wspec,
            pl.BlockSpec((4, D), lambda i: (0, 0)),
        ],
        out_specs=[pl.BlockSpec((blk, D), lambda i: (i, 0))] * 4,
        out_shape=[jax.ShapeDtypeStruct((N, D), jnp.float32)] * 4,
    )(x, wqt, wkt, wvt, wst, bstack)


def _edge_body(q_hbm, k_hbm, v_hbm, src_hbm, dst_hbm, a0_hbm, a1_hbm, wet_hbm,
               zero_hbm, part_hbm, w_hbm,
               acc_sh, qb, kb, vb, msgb, wqb, srcb, dstb, a0b, a1b, wetb, sem):
    c = lax.axis_index("c")
    s = lax.axis_index("s")
    wid = c * NS + s
    ebase = wid * E_PER_W
    rbase = s * ROWS_PER_TILE

    pltpu.sync_copy(zero_hbm.at[pl.ds(rbase, ROWS_PER_TILE)],
                    acc_sh.at[pl.ds(rbase, ROWS_PER_TILE)])
    pltpu.sync_copy(wet_hbm, wetb)
    plsc.subcore_barrier()

    wet0 = [wetb[0, pl.ds(16 * l, 16)] for l in range(8)]
    wet1 = [wetb[1, pl.ds(16 * l, 16)] for l in range(8)]
    lane = lax.iota(jnp.int32, 16)
    lane0 = lane < 1

    def chunk_body(j, _):
        off = ebase + j * CHUNK
        pltpu.sync_copy(src_hbm.at[pl.ds(off, CHUNK)], srcb)
        pltpu.sync_copy(dst_hbm.at[pl.ds(off, CHUNK)], dstb)
        pltpu.sync_copy(a0_hbm.at[pl.ds(off, CHUNK)], a0b)
        pltpu.sync_copy(a1_hbm.at[pl.ds(off, CHUNK)], a1b)
        pltpu.async_copy(q_hbm.at[dstb], qb, sem).wait()
        pltpu.async_copy(k_hbm.at[srcb], kb, sem).wait()
        pltpu.async_copy(v_hbm.at[srcb], vb, sem).wait()

        def edge_body(i, _):
            ibc = jnp.full((16,), i, dtype=jnp.int32)
            a0 = plsc.load_gather(a0b, [ibc])
            a1 = plsc.load_gather(a1b, [ibc])
            ev = [a0 * wet0[l] + a1 * wet1[l] for l in range(8)]
            pr = [qb[i, pl.ds(16 * l, 16)] * (kb[i, pl.ds(16 * l, 16)] + ev[l])
                  for l in range(8)]
            wb = []
            for h in range(H):
                sh = jnp.sum(pr[2 * h] + pr[2 * h + 1]) * _RS32
                whv = jnp.exp(jnp.full((16,), sh, dtype=jnp.float32))
                wb.append(whv)
                plsc.store_scatter(wqb, [ibc * H + h], whv, mask=lane0)
            for l in range(8):
                msgb[i, pl.ds(16 * l, 16)] = (vb[i, pl.ds(16 * l, 16)] + ev[l]) * wb[l // 2]
            return 0

        lax.fori_loop(0, CHUNK, edge_body, 0)
        pltpu.sync_copy(msgb, acc_sh.at[dstb], add=True)
        pltpu.sync_copy(wqb, w_hbm.at[pl.ds(off * H, CHUNK * H)])
        return 0

    lax.fori_loop(0, NCHUNK, chunk_body, 0)
    plsc.subcore_barrier()
    pltpu.sync_copy(acc_sh.at[pl.ds(rbase, ROWS_PER_TILE)],
                    part_hbm.at[c, pl.ds(rbase, ROWS_PER_TILE)])


def _edge_pass(q, k, v, src, dst, a0, a1, wet, zeros):
    mesh = plsc.VectorSubcoreMesh(core_axis_name="c", subcore_axis_name="s")
    f = pl.kernel(
        _edge_body,
        mesh=mesh,
        compiler_params=_SC_PARAMS,
        out_type=(
            jax.ShapeDtypeStruct((NC, NPAD, D), jnp.float32),
            jax.ShapeDtypeStruct((H * E,), jnp.float32),
        ),
        scratch_types=[
            pltpu.VMEM_SHARED((NPAD, D), jnp.float32),
            pltpu.VMEM((CHUNK, D), jnp.float32),
            pltpu.VMEM((CHUNK, D), jnp.float32),
            pltpu.VMEM((CHUNK, D), jnp.float32),
            pltpu.VMEM((CHUNK, D), jnp.float32),
            pltpu.VMEM((H * CHUNK,), jnp.float32),
            pltpu.VMEM((CHUNK,), jnp.int32),
            pltpu.VMEM((CHUNK,), jnp.int32),
            pltpu.VMEM((CHUNK,), jnp.float32),
            pltpu.VMEM((CHUNK,), jnp.float32),
            pltpu.VMEM((2, D), jnp.float32),
            pltpu.SemaphoreType.DMA,
        ],
    )
    return f(q, k, v, src, dst, a0, a1, wet, zeros)


def _ssum_body(w_hbm, dst_hbm, zero_hbm, out_hbm, ssumb, wcb, dstb):
    c = lax.axis_index("c")
    s = lax.axis_index("s")
    wid = c * NS + s
    ebase = wid * E_PER_W

    pltpu.sync_copy(zero_hbm.at[pl.ds(0, SSROWS)], ssumb)
    lane = lax.iota(jnp.int32, 16)
    lane4 = lane & 3
    m4 = lane < 4

    def chunk_body(j, _):
        off = ebase + j * CHUNK
        pltpu.sync_copy(dst_hbm.at[pl.ds(off, CHUNK)], dstb)
        pltpu.sync_copy(w_hbm.at[pl.ds(off * H, CHUNK * H)], wcb)

        def edge_body(i, _):
            ibc = jnp.full((16,), i, dtype=jnp.int32)
            d = plsc.load_gather(dstb, [ibc])
            wrep = plsc.load_gather(wcb, [ibc * H + lane4])
            flat = d * H + lane4
            plsc.addupdate_scatter(ssumb, [flat >> 7, flat & 127], wrep, mask=m4)
            return 0

        lax.fori_loop(0, CHUNK, edge_body, 0)
        return 0

    lax.fori_loop(0, NCHUNK, chunk_body, 0)
    pltpu.sync_copy(ssumb, out_hbm.at[wid])


def _ssum_pass(w, dst, zeros):
    mesh = plsc.VectorSubcoreMesh(core_axis_name="c", subcore_axis_name="s")
    f = pl.kernel(
        _ssum_body,
        mesh=mesh,
        compiler_params=_SC_PARAMS,
        out_type=jax.ShapeDtypeStruct((NW, SSROWS, D), jnp.float32),
        scratch_types=[
            pltpu.VMEM((SSROWS, D), jnp.float32),
            pltpu.VMEM((H * CHUNK,), jnp.float32),
            pltpu.VMEM((CHUNK,), jnp.int32),
        ],
    )
    return f(w, dst, zeros)


def _combine_body(p_ref, p2_ref, s_ref, out_ref, stats_ref, acc2):
    i = pl.program_id(0)
    p = p_ref[...]
    accs = p[0] + p[1]
    ws = jnp.sum(p2_ref[...], axis=0)
    blk = ws.shape[0]
    denom = jnp.reshape(jnp.broadcast_to(ws[:, :, None], (blk, H, C)), (blk, D))
    op = accs / (denom + 1e-16) + s_ref[...]
    out_ref[...] = op

    @pl.when(i == 0)
    def _():
        acc2[...] = jnp.zeros_like(acc2)

    acc2[0, :] += jnp.sum(op, axis=0)
    acc2[1, :] += jnp.sum(op * op, axis=0)

    @pl.when(i == pl.num_programs(0) - 1)
    def _():
        stats_ref[...] = acc2[...]


def _combine(parts, parts2, sproj):
    blk = 400
    grid = N // blk
    return pl.pallas_call(
        _combine_body,
        grid=(grid,),
        in_specs=[
            pl.BlockSpec((NC, blk, D), lambda i: (0, i, 0)),
            pl.BlockSpec((NW, blk, H), lambda i: (0, i, 0)),
            pl.BlockSpec((blk, D), lambda i: (i, 0)),
        ],
        out_specs=[
            pl.BlockSpec((blk, D), lambda i: (i, 0)),
            pl.BlockSpec((2, D), lambda i: (0, 0)),
        ],
        out_shape=[
            jax.ShapeDtypeStruct((N, D), jnp.float32),
            jax.ShapeDtypeStruct((2, D), jnp.float32),
        ],
        scratch_shapes=[pltpu.VMEM((2, D), jnp.float32)],
    )(parts, parts2, sproj)


def _bn_body(op_ref, stats_ref, g_ref, b_ref, y_ref):
    st = stats_ref[...]
    mu = st[0:1, :] * (1.0 / N)
    var = st[1:2, :] * (1.0 / N) - mu * mu
    inv = lax.rsqrt(var + 1e-5)
    y = (op_ref[...] - mu) * (inv * g_ref[...]) + b_ref[...]
    y_ref[...] = jnp.where(y > 0, y, 0.01 * y)


def _batchnorm(op, stats, gamma, beta):
    blk = 400
    grid = N // blk
    return pl.pallas_call(
        _bn_body,
        grid=(grid,),
        in_specs=[
            pl.BlockSpec((blk, D), lambda i: (i, 0)),
            pl.BlockSpec((2, D), lambda i: (0, 0)),
            pl.BlockSpec((1, D), lambda i: (0, 0)),
            pl.BlockSpec((1, D), lambda i: (0, 0)),
        ],
        out_specs=pl.BlockSpec((blk, D), lambda i: (i, 0)),
        out_shape=jax.ShapeDtypeStruct((N, D), jnp.float32),
    )(op, stats, gamma, beta)


def kernel(x, edge_idx, edge_attr, Wq, bq, Wk, bk, Wv, bv, We, Ws, bs, gamma, beta):
    src = edge_idx[0].astype(jnp.int32)
    dst = edge_idx[1].astype(jnp.int32)
    bstack = jnp.stack([bq, bk, bv, bs])
    q, k, v, sproj = _projections(x, Wq.T, Wk.T, Wv.T, Ws.T, bstack)
    wet = We.T
    a0 = edge_attr[:, 0]
    a1 = edge_attr[:, 1]
    zeros = jnp.zeros((NPAD, D), jnp.float32)
    parts, w = _edge_pass(q, k, v, src, dst, a0, a1, wet, zeros)
    parts2 = _ssum_pass(w, dst, zeros).reshape(NW, NPAD, H)
    op, stats = _combine(parts[:, :N], parts2[:, :N], sproj)
    return _batchnorm(op, stats, gamma.reshape(1, D), beta.reshape(1, D))


# speedup vs baseline: 19.8163x; 1.3653x over previous
import math

import jax
import jax.numpy as jnp
from jax import lax
from jax.experimental import pallas as pl
from jax.experimental.pallas import tpu as pltpu
from jax.experimental.pallas import tpu_sc as plsc

N = 10000
E = 320000
F = 128
H = 4
C = 32
D = H * C

NC = 2
NS = 16
NW = NC * NS
E_PER_W = E // NW
CHUNK = 80
NCHUNK = E_PER_W // CHUNK
NPAD = 10240
ROWS_PER_TILE = NPAD // NS
SSROWS = NPAD * H // D

_RS32 = 1.0 / math.sqrt(C)

_SC_PARAMS = pltpu.CompilerParams(needs_layout_passes=False)


def _proj_body(x_ref, wq_ref, wk_ref, wv_ref, ws_ref, b_ref,
               q_ref, k_ref, v_ref, s_ref):
    xb = x_ref[...]
    b = b_ref[...]
    q_ref[...] = jnp.dot(xb, wq_ref[...], preferred_element_type=jnp.float32) + b[0:1]
    k_ref[...] = jnp.dot(xb, wk_ref[...], preferred_element_type=jnp.float32) + b[1:2]
    v_ref[...] = jnp.dot(xb, wv_ref[...], preferred_element_type=jnp.float32) + b[2:3]
    s_ref[...] = jnp.dot(xb, ws_ref[...], preferred_element_type=jnp.float32) + b[3:4]


def _projections(x, wqt, wkt, wvt, wst, bstack):
    blk = 400
    grid = N // blk
    wspec = pl.BlockSpec((F, D), lambda i: (0, 0))
    return pl.pallas_call(
        _proj_body,
        grid=(grid,),
        in_specs=[
            pl.BlockSpec((blk, F), lambda i: (i, 0)),
            wspec, wspec, wspec, wspec,
            pl.BlockSpec((4, D), lambda i: (0, 0)),
        ],
        out_specs=[pl.BlockSpec((blk, D), lambda i: (i, 0))] * 4,
        out_shape=[jax.ShapeDtypeStruct((N, D), jnp.float32)] * 4,
    )(x, wqt, wkt, wvt, wst, bstack)


def _edge_body(q_hbm, k_hbm, v_hbm, src_hbm, dst_hbm, a0_hbm, a1_hbm, wet_hbm,
               zero_hbm, part_hbm, w_hbm,
               acc_sh, qb, kb, vb, msgb, wqb, srcb, dstb, a0b, a1b, wetb, sem):
    c = lax.axis_index("c")
    s = lax.axis_index("s")
    wid = c * NS + s
    ebase = wid * E_PER_W
    rbase = s * ROWS_PER_TILE

    pltpu.sync_copy(zero_hbm.at[pl.ds(rbase, ROWS_PER_TILE)],
                    acc_sh.at[pl.ds(rbase, ROWS_PER_TILE)])
    pltpu.sync_copy(wet_hbm, wetb)
    plsc.subcore_barrier()

    wet0 = [wetb[0, pl.ds(16 * l, 16)] for l in range(8)]
    wet1 = [wetb[1, pl.ds(16 * l, 16)] for l in range(8)]
    lane = lax.iota(jnp.int32, 16)
    lane0 = lane < 1

    def chunk_body(j, _):
        off = ebase + j * CHUNK
        pltpu.sync_copy(src_hbm.at[pl.ds(off, CHUNK)], srcb)
        pltpu.sync_copy(dst_hbm.at[pl.ds(off, CHUNK)], dstb)
        pltpu.sync_copy(a0_hbm.at[pl.ds(off, CHUNK)], a0b)
        pltpu.sync_copy(a1_hbm.at[pl.ds(off, CHUNK)], a1b)
        pltpu.async_copy(q_hbm.at[dstb], qb, sem).wait()
        pltpu.async_copy(k_hbm.at[srcb], kb, sem).wait()
        pltpu.async_copy(v_hbm.at[srcb], vb, sem).wait()

        @plsc.parallel_loop(0, CHUNK, unroll=2)
        def edge_body(i):
            ibc = jnp.full((16,), i, dtype=jnp.int32)
            a0 = plsc.load_gather(a0b, [ibc])
            a1 = plsc.load_gather(a1b, [ibc])
            ev = [a0 * wet0[l] + a1 * wet1[l] for l in range(8)]
            pr = [qb[i, pl.ds(16 * l, 16)] * (kb[i, pl.ds(16 * l, 16)] + ev[l])
                  for l in range(8)]
            wb = []
            for h in range(H):
                sh = jnp.sum(pr[2 * h] + pr[2 * h + 1]) * _RS32
                whv = jnp.exp(jnp.full((16,), sh, dtype=jnp.float32))
                wb.append(whv)
                plsc.store_scatter(wqb, [ibc * H + h], whv, mask=lane0)
            for l in range(8):
                msgb[i, pl.ds(16 * l, 16)] = (vb[i, pl.ds(16 * l, 16)] + ev[l]) * wb[l // 2]
        pltpu.sync_copy(msgb, acc_sh.at[dstb], add=True)
        pltpu.sync_copy(wqb, w_hbm.at[pl.ds(off * H, CHUNK * H)])
        return 0

    lax.fori_loop(0, NCHUNK, chunk_body, 0)
    plsc.subcore_barrier()
    pltpu.sync_copy(acc_sh.at[pl.ds(rbase, ROWS_PER_TILE)],
                    part_hbm.at[c, pl.ds(rbase, ROWS_PER_TILE)])


def _edge_pass(q, k, v, src, dst, a0, a1, wet, zeros):
    mesh = plsc.VectorSubcoreMesh(core_axis_name="c", subcore_axis_name="s")
    f = pl.kernel(
        _edge_body,
        mesh=mesh,
        compiler_params=_SC_PARAMS,
        out_type=(
            jax.ShapeDtypeStruct((NC, NPAD, D), jnp.float32),
            jax.ShapeDtypeStruct((H * E,), jnp.float32),
        ),
        scratch_types=[
            pltpu.VMEM_SHARED((NPAD, D), jnp.float32),
            pltpu.VMEM((CHUNK, D), jnp.float32),
            pltpu.VMEM((CHUNK, D), jnp.float32),
            pltpu.VMEM((CHUNK, D), jnp.float32),
            pltpu.VMEM((CHUNK, D), jnp.float32),
            pltpu.VMEM((H * CHUNK,), jnp.float32),
            pltpu.VMEM((CHUNK,), jnp.int32),
            pltpu.VMEM((CHUNK,), jnp.int32),
            pltpu.VMEM((CHUNK,), jnp.float32),
            pltpu.VMEM((CHUNK,), jnp.float32),
            pltpu.VMEM((2, D), jnp.float32),
            pltpu.SemaphoreType.DMA,
        ],
    )
    return f(q, k, v, src, dst, a0, a1, wet, zeros)


def _ssum_body(w_hbm, dst_hbm, zero_hbm, out_hbm, ssumb, wcb, dstb):
    c = lax.axis_index("c")
    s = lax.axis_index("s")
    wid = c * NS + s
    ebase = wid * E_PER_W

    pltpu.sync_copy(zero_hbm.at[pl.ds(0, SSROWS)], ssumb)
    lane = lax.iota(jnp.int32, 16)
    lane4 = lane & 3
    m4 = lane < 4

    def chunk_body(j, _):
        off = ebase + j * CHUNK
        pltpu.sync_copy(dst_hbm.at[pl.ds(off, CHUNK)], dstb)
        pltpu.sync_copy(w_hbm.at[pl.ds(off * H, CHUNK * H)], wcb)

        @plsc.parallel_loop(0, CHUNK, unroll=4)
        def edge_body(i):
            ibc = jnp.full((16,), i, dtype=jnp.int32)
            d = plsc.load_gather(dstb, [ibc])
            wrep = plsc.load_gather(wcb, [ibc * H + lane4])
            flat = d * H + lane4
            plsc.addupdate_scatter(ssumb, [flat >> 7, flat & 127], wrep, mask=m4)

        return 0

    lax.fori_loop(0, NCHUNK, chunk_body, 0)
    pltpu.sync_copy(ssumb, out_hbm.at[wid])


def _ssum_pass(w, dst, zeros):
    mesh = plsc.VectorSubcoreMesh(core_axis_name="c", subcore_axis_name="s")
    f = pl.kernel(
        _ssum_body,
        mesh=mesh,
        compiler_params=_SC_PARAMS,
        out_type=jax.ShapeDtypeStruct((NW, SSROWS, D), jnp.float32),
        scratch_types=[
            pltpu.VMEM((SSROWS, D), jnp.float32),
            pltpu.VMEM((H * CHUNK,), jnp.float32),
            pltpu.VMEM((CHUNK,), jnp.int32),
        ],
    )
    return f(w, dst, zeros)


def _combine_body(p_ref, p2_ref, s_ref, out_ref, stats_ref, acc2):
    i = pl.program_id(0)
    p = p_ref[...]
    accs = p[0] + p[1]
    ws = jnp.sum(p2_ref[...], axis=0)
    blk = ws.shape[0]
    denom = jnp.reshape(jnp.broadcast_to(ws[:, :, None], (blk, H, C)), (blk, D))
    op = accs / (denom + 1e-16) + s_ref[...]
    out_ref[...] = op

    @pl.when(i == 0)
    def _():
        acc2[...] = jnp.zeros_like(acc2)

    acc2[0, :] += jnp.sum(op, axis=0)
    acc2[1, :] += jnp.sum(op * op, axis=0)

    @pl.when(i == pl.num_programs(0) - 1)
    def _():
        stats_ref[...] = acc2[...]


def _combine(parts, parts2, sproj):
    blk = 400
    grid = N // blk
    return pl.pallas_call(
        _combine_body,
        grid=(grid,),
        in_specs=[
            pl.BlockSpec((NC, blk, D), lambda i: (0, i, 0)),
            pl.BlockSpec((NW, blk, H), lambda i: (0, i, 0)),
            pl.BlockSpec((blk, D), lambda i: (i, 0)),
        ],
        out_specs=[
            pl.BlockSpec((blk, D), lambda i: (i, 0)),
            pl.BlockSpec((2, D), lambda i: (0, 0)),
        ],
        out_shape=[
            jax.ShapeDtypeStruct((N, D), jnp.float32),
            jax.ShapeDtypeStruct((2, D), jnp.float32),
        ],
        scratch_shapes=[pltpu.VMEM((2, D), jnp.float32)],
    )(parts, parts2, sproj)


def _bn_body(op_ref, stats_ref, g_ref, b_ref, y_ref):
    st = stats_ref[...]
    mu = st[0:1, :] * (1.0 / N)
    var = st[1:2, :] * (1.0 / N) - mu * mu
    inv = lax.rsqrt(var + 1e-5)
    y = (op_ref[...] - mu) * (inv * g_ref[...]) + b_ref[...]
    y_ref[...] = jnp.where(y > 0, y, 0.01 * y)


def _batchnorm(op, stats, gamma, beta):
    blk = 400
    grid = N // blk
    return pl.pallas_call(
        _bn_body,
        grid=(grid,),
        in_specs=[
            pl.BlockSpec((blk, D), lambda i: (i, 0)),
            pl.BlockSpec((2, D), lambda i: (0, 0)),
            pl.BlockSpec((1, D), lambda i: (0, 0)),
            pl.BlockSpec((1, D), lambda i: (0, 0)),
        ],
        out_specs=pl.BlockSpec((blk, D), lambda i: (i, 0)),
        out_shape=jax.ShapeDtypeStruct((N, D), jnp.float32),
    )(op, stats, gamma, beta)


def kernel(x, edge_idx, edge_attr, Wq, bq, Wk, bk, Wv, bv, We, Ws, bs, gamma, beta):
    src = edge_idx[0].astype(jnp.int32)
    dst = edge_idx[1].astype(jnp.int32)
    bstack = jnp.stack([bq, bk, bv, bs])
    q, k, v, sproj = _projections(x, Wq.T, Wk.T, Wv.T, Ws.T, bstack)
    wet = We.T
    a0 = edge_attr[:, 0]
    a1 = edge_attr[:, 1]
    zeros = jnp.zeros((NPAD, D), jnp.float32)
    parts, w = _edge_pass(q, k, v, src, dst, a0, a1, wet, zeros)
    parts2 = _ssum_pass(w, dst, zeros).reshape(NW, NPAD, H)
    op, stats = _combine(parts[:, :N], parts2[:, :N], sproj)
    return _batchnorm(op, stats, gamma.reshape(1, D), beta.reshape(1, D))


# speedup vs baseline: 20.6427x; 1.0417x over previous
import math

import jax
import jax.numpy as jnp
from jax import lax
from jax.experimental import pallas as pl
from jax.experimental.pallas import tpu as pltpu
from jax.experimental.pallas import tpu_sc as plsc

N = 10000
E = 320000
F = 128
H = 4
C = 32
D = H * C

NC = 2
NS = 16
NW = NC * NS
E_PER_W = E // NW
CHUNK = 80
NCHUNK = E_PER_W // CHUNK
NPAD = 10240
ROWS_PER_TILE = NPAD // NS
SSROWS = NPAD * H // D

_RS32 = 1.0 / math.sqrt(C)

_SC_PARAMS = pltpu.CompilerParams(needs_layout_passes=False)


def _proj_body(x_ref, wq_ref, wk_ref, wv_ref, ws_ref, b_ref,
               q_ref, k_ref, v_ref, s_ref):
    xb = x_ref[...]
    b = b_ref[...]
    q_ref[...] = jnp.dot(xb, wq_ref[...], preferred_element_type=jnp.float32) + b[0:1]
    k_ref[...] = jnp.dot(xb, wk_ref[...], preferred_element_type=jnp.float32) + b[1:2]
    v_ref[...] = jnp.dot(xb, wv_ref[...], preferred_element_type=jnp.float32) + b[2:3]
    s_ref[...] = jnp.dot(xb, ws_ref[...], preferred_element_type=jnp.float32) + b[3:4]


def _projections(x, wqt, wkt, wvt, wst, bstack):
    blk = 400
    grid = N // blk
    wspec = pl.BlockSpec((F, D), lambda i: (0, 0))
    return pl.pallas_call(
        _proj_body,
        grid=(grid,),
        in_specs=[
            pl.BlockSpec((blk, F), lambda i: (i, 0)),
            wspec, wspec, wspec, wspec,
            pl.BlockSpec((4, D), lambda i: (0, 0)),
        ],
        out_specs=[pl.BlockSpec((blk, D), lambda i: (i, 0))] * 4,
        out_shape=[jax.ShapeDtypeStruct((N, D), jnp.float32)] * 4,
    )(x, wqt, wkt, wvt, wst, bstack)


def _edge_body(q_hbm, k_hbm, v_hbm, src_hbm, dst_hbm, a0_hbm, a1_hbm, wet_hbm,
               zero_hbm, part_hbm, w_hbm,
               acc_sh, qb, kb, vb, msgb, wqb, srcb, dstb, a0b, a1b, wetb, sem):
    c = lax.axis_index("c")
    s = lax.axis_index("s")
    wid = c * NS + s
    ebase = wid * E_PER_W
    rbase = s * ROWS_PER_TILE

    pltpu.sync_copy(zero_hbm.at[pl.ds(rbase, ROWS_PER_TILE)],
                    acc_sh.at[pl.ds(rbase, ROWS_PER_TILE)])
    pltpu.sync_copy(wet_hbm, wetb)
    plsc.subcore_barrier()

    wet0 = [wetb[0, pl.ds(16 * l, 16)] for l in range(8)]
    wet1 = [wetb[1, pl.ds(16 * l, 16)] for l in range(8)]
    lane = lax.iota(jnp.int32, 16)
    lane0 = lane < 1

    def chunk_body(j, _):
        off = ebase + j * CHUNK
        pltpu.sync_copy(src_hbm.at[pl.ds(off, CHUNK)], srcb)
        pltpu.sync_copy(dst_hbm.at[pl.ds(off, CHUNK)], dstb)
        pltpu.sync_copy(a0_hbm.at[pl.ds(off, CHUNK)], a0b)
        pltpu.sync_copy(a1_hbm.at[pl.ds(off, CHUNK)], a1b)
        pltpu.async_copy(q_hbm.at[dstb], qb, sem).wait()
        pltpu.async_copy(k_hbm.at[srcb], kb, sem).wait()
        pltpu.async_copy(v_hbm.at[srcb], vb, sem).wait()

        @plsc.parallel_loop(0, CHUNK, unroll=4)
        def edge_body(i):
            ibc = jnp.full((16,), i, dtype=jnp.int32)
            a0 = plsc.load_gather(a0b, [ibc])
            a1 = plsc.load_gather(a1b, [ibc])
            ev = [a0 * wet0[l] + a1 * wet1[l] for l in range(8)]
            pr = [qb[i, pl.ds(16 * l, 16)] * (kb[i, pl.ds(16 * l, 16)] + ev[l])
                  for l in range(8)]
            wb = []
            for h in range(H):
                sh = jnp.sum(pr[2 * h] + pr[2 * h + 1]) * _RS32
                whv = jnp.exp(jnp.full((16,), sh, dtype=jnp.float32))
                wb.append(whv)
                plsc.store_scatter(wqb, [ibc * H + h], whv, mask=lane0)
            for l in range(8):
                msgb[i, pl.ds(16 * l, 16)] = (vb[i, pl.ds(16 * l, 16)] + ev[l]) * wb[l // 2]
        pltpu.sync_copy(msgb, acc_sh.at[dstb], add=True)
        pltpu.sync_copy(wqb, w_hbm.at[pl.ds(off * H, CHUNK * H)])
        return 0

    lax.fori_loop(0, NCHUNK, chunk_body, 0)
    plsc.subcore_barrier()
    pltpu.sync_copy(acc_sh.at[pl.ds(rbase, ROWS_PER_TILE)],
                    part_hbm.at[c, pl.ds(rbase, ROWS_PER_TILE)])


def _edge_pass(q, k, v, src, dst, a0, a1, wet, zeros):
    mesh = plsc.VectorSubcoreMesh(core_axis_name="c", subcore_axis_name="s")
    f = pl.kernel(
        _edge_body,
        mesh=mesh,
        compiler_params=_SC_PARAMS,
        out_type=(
            jax.ShapeDtypeStruct((NC, NPAD, D), jnp.float32),
            jax.ShapeDtypeStruct((H * E,), jnp.float32),
        ),
        scratch_types=[
            pltpu.VMEM_SHARED((NPAD, D), jnp.float32),
            pltpu.VMEM((CHUNK, D), jnp.float32),
            pltpu.VMEM((CHUNK, D), jnp.float32),
            pltpu.VMEM((CHUNK, D), jnp.float32),
            pltpu.VMEM((CHUNK, D), jnp.float32),
            pltpu.VMEM((H * CHUNK,), jnp.float32),
            pltpu.VMEM((CHUNK,), jnp.int32),
            pltpu.VMEM((CHUNK,), jnp.int32),
            pltpu.VMEM((CHUNK,), jnp.float32),
            pltpu.VMEM((CHUNK,), jnp.float32),
            pltpu.VMEM((2, D), jnp.float32),
            pltpu.SemaphoreType.DMA,
        ],
    )
    return f(q, k, v, src, dst, a0, a1, wet, zeros)


def _ssum_body(w_hbm, dst_hbm, zero_hbm, out_hbm, ssumb, wcb, dstb):
    c = lax.axis_index("c")
    s = lax.axis_index("s")
    wid = c * NS + s
    ebase = wid * E_PER_W

    pltpu.sync_copy(zero_hbm.at[pl.ds(0, SSROWS)], ssumb)
    lane = lax.iota(jnp.int32, 16)
    lane4 = lane & 3
    m4 = lane < 4

    def chunk_body(j, _):
        off = ebase + j * CHUNK
        pltpu.sync_copy(dst_hbm.at[pl.ds(off, CHUNK)], dstb)
        pltpu.sync_copy(w_hbm.at[pl.ds(off * H, CHUNK * H)], wcb)

        @plsc.parallel_loop(0, CHUNK, unroll=4)
        def edge_body(i):
            ibc = jnp.full((16,), i, dtype=jnp.int32)
            d = plsc.load_gather(dstb, [ibc])
            wrep = plsc.load_gather(wcb, [ibc * H + lane4])
            flat = d * H + lane4
            plsc.addupdate_scatter(ssumb, [flat >> 7, flat & 127], wrep, mask=m4)

        return 0

    lax.fori_loop(0, NCHUNK, chunk_body, 0)
    pltpu.sync_copy(ssumb, out_hbm.at[wid])


def _ssum_pass(w, dst, zeros):
    mesh = plsc.VectorSubcoreMesh(core_axis_name="c", subcore_axis_name="s")
    f = pl.kernel(
        _ssum_body,
        mesh=mesh,
        compiler_params=_SC_PARAMS,
        out_type=jax.ShapeDtypeStruct((NW, SSROWS, D), jnp.float32),
        scratch_types=[
            pltpu.VMEM((SSROWS, D), jnp.float32),
            pltpu.VMEM((H * CHUNK,), jnp.float32),
            pltpu.VMEM((CHUNK,), jnp.int32),
        ],
    )
    return f(w, dst, zeros)


def _combine_body(p_ref, p2_ref, s_ref, out_ref, stats_ref, acc2):
    i = pl.program_id(0)
    p = p_ref[...]
    accs = p[0] + p[1]
    ws = jnp.sum(p2_ref[...], axis=0)
    blk = ws.shape[0]
    denom = jnp.reshape(jnp.broadcast_to(ws[:, :, None], (blk, H, C)), (blk, D))
    op = accs / (denom + 1e-16) + s_ref[...]
    out_ref[...] = op

    @pl.when(i == 0)
    def _():
        acc2[...] = jnp.zeros_like(acc2)

    acc2[0, :] += jnp.sum(op, axis=0)
    acc2[1, :] += jnp.sum(op * op, axis=0)

    @pl.when(i == pl.num_programs(0) - 1)
    def _():
        stats_ref[...] = acc2[...]


def _combine(parts, parts2, sproj):
    blk = 400
    grid = N // blk
    return pl.pallas_call(
        _combine_body,
        grid=(grid,),
        in_specs=[
            pl.BlockSpec((NC, blk, D), lambda i: (0, i, 0)),
            pl.BlockSpec((NW, blk, H), lambda i: (0, i, 0)),
            pl.BlockSpec((blk, D), lambda i: (i, 0)),
        ],
        out_specs=[
            pl.BlockSpec((blk, D), lambda i: (i, 0)),
            pl.BlockSpec((2, D), lambda i: (0, 0)),
        ],
        out_shape=[
            jax.ShapeDtypeStruct((N, D), jnp.float32),
            jax.ShapeDtypeStruct((2, D), jnp.float32),
        ],
        scratch_shapes=[pltpu.VMEM((2, D), jnp.float32)],
    )(parts, parts2, sproj)


def _bn_body(op_ref, stats_ref, g_ref, b_ref, y_ref):
    st = stats_ref[...]
    mu = st[0:1, :] * (1.0 / N)
    var = st[1:2, :] * (1.0 / N) - mu * mu
    inv = lax.rsqrt(var + 1e-5)
    y = (op_ref[...] - mu) * (inv * g_ref[...]) + b_ref[...]
    y_ref[...] = jnp.where(y > 0, y, 0.01 * y)


def _batchnorm(op, stats, gamma, beta):
    blk = 400
    grid = N // blk
    return pl.pallas_call(
        _bn_body,
        grid=(grid,),
        in_specs=[
            pl.BlockSpec((blk, D), lambda i: (i, 0)),
            pl.BlockSpec((2, D), lambda i: (0, 0)),
            pl.BlockSpec((1, D), lambda i: (0, 0)),
            pl.BlockSpec((1, D), lambda i: (0, 0)),
        ],
        out_specs=pl.BlockSpec((blk, D), lambda i: (i, 0)),
        out_shape=jax.ShapeDtypeStruct((N, D), jnp.float32),
    )(op, stats, gamma, beta)


def kernel(x, edge_idx, edge_attr, Wq, bq, Wk, bk, Wv, bv, We, Ws, bs, gamma, beta):
    src = edge_idx[0].astype(jnp.int32)
    dst = edge_idx[1].astype(jnp.int32)
    bstack = jnp.stack([bq, bk, bv, bs])
    q, k, v, sproj = _projections(x, Wq.T, Wk.T, Wv.T, Ws.T, bstack)
    wet = We.T
    a0 = edge_attr[:, 0]
    a1 = edge_attr[:, 1]
    zeros = jnp.zeros((NPAD, D), jnp.float32)
    parts, w = _edge_pass(q, k, v, src, dst, a0, a1, wet, zeros)
    parts2 = _ssum_pass(w, dst, zeros).reshape(NW, NPAD, H)
    op, stats = _combine(parts[:, :N], parts2[:, :N], sproj)
    return _batchnorm(op, stats, gamma.reshape(1, D), beta.reshape(1, D))


# speedup vs baseline: 21.2031x; 1.0271x over previous
import math

import jax
import jax.numpy as jnp
from jax import lax
from jax.experimental import pallas as pl
from jax.experimental.pallas import tpu as pltpu
from jax.experimental.pallas import tpu_sc as plsc

N = 10000
E = 320000
F = 128
H = 4
C = 32
D = H * C

NC = 2
NS = 16
NW = NC * NS
E_PER_W = E // NW
CHUNK = 80
NCHUNK = E_PER_W // CHUNK
NPAD = 10240
ROWS_PER_TILE = NPAD // NS
SSROWS = NPAD * H // D

_RS32 = 1.0 / math.sqrt(C)

_SC_PARAMS = pltpu.CompilerParams(needs_layout_passes=False)


def _proj_body(x_ref, wq_ref, wk_ref, wv_ref, ws_ref, b_ref,
               q_ref, k_ref, v_ref, s_ref):
    xb = x_ref[...]
    b = b_ref[...]
    q_ref[...] = jnp.dot(xb, wq_ref[...], preferred_element_type=jnp.float32) + b[0:1]
    k_ref[...] = jnp.dot(xb, wk_ref[...], preferred_element_type=jnp.float32) + b[1:2]
    v_ref[...] = jnp.dot(xb, wv_ref[...], preferred_element_type=jnp.float32) + b[2:3]
    s_ref[...] = jnp.dot(xb, ws_ref[...], preferred_element_type=jnp.float32) + b[3:4]


def _projections(x, wqt, wkt, wvt, wst, bstack):
    blk = 400
    grid = N // blk
    wspec = pl.BlockSpec((F, D), lambda i: (0, 0))
    return pl.pallas_call(
        _proj_body,
        grid=(grid,),
        in_specs=[
            pl.BlockSpec((blk, F), lambda i: (i, 0)),
            wspec, wspec, wspec, wspec,
            pl.BlockSpec((4, D), lambda i: (0, 0)),
        ],
        out_specs=[pl.BlockSpec((blk, D), lambda i: (i, 0))] * 4,
        out_shape=[jax.ShapeDtypeStruct((N, D), jnp.float32)] * 4,
    )(x, wqt, wkt, wvt, wst, bstack)


def _edge_body(q_hbm, k_hbm, v_hbm, src_hbm, dst_hbm, a0_hbm, a1_hbm, wet_hbm,
               zero_hbm, part_hbm, w_hbm,
               acc_sh, qb, kb, vb, msgb, wqb, srcb, dstb, a0b, a1b, wetb, sem):
    c = lax.axis_index("c")
    s = lax.axis_index("s")
    wid = c * NS + s
    ebase = wid * E_PER_W
    rbase = s * ROWS_PER_TILE

    pltpu.sync_copy(zero_hbm.at[pl.ds(rbase, ROWS_PER_TILE)],
                    acc_sh.at[pl.ds(rbase, ROWS_PER_TILE)])
    pltpu.sync_copy(wet_hbm, wetb)
    plsc.subcore_barrier()

    wet0 = [wetb[0, pl.ds(16 * l, 16)] for l in range(8)]
    wet1 = [wetb[1, pl.ds(16 * l, 16)] for l in range(8)]
    lane = lax.iota(jnp.int32, 16)
    lane0 = lane < 1

    def chunk_body(j, _):
        off = ebase + j * CHUNK
        pltpu.sync_copy(src_hbm.at[pl.ds(off, CHUNK)], srcb)
        pltpu.sync_copy(dst_hbm.at[pl.ds(off, CHUNK)], dstb)
        pltpu.sync_copy(a0_hbm.at[pl.ds(off, CHUNK)], a0b)
        pltpu.sync_copy(a1_hbm.at[pl.ds(off, CHUNK)], a1b)
        pltpu.async_copy(q_hbm.at[dstb], qb, sem).wait()
        pltpu.async_copy(k_hbm.at[srcb], kb, sem).wait()
        pltpu.async_copy(v_hbm.at[srcb], vb, sem).wait()

        @plsc.parallel_loop(0, CHUNK, unroll=8)
        def edge_body(i):
            ibc = jnp.full((16,), i, dtype=jnp.int32)
            a0 = plsc.load_gather(a0b, [ibc])
            a1 = plsc.load_gather(a1b, [ibc])
            ev = [a0 * wet0[l] + a1 * wet1[l] for l in range(8)]
            pr = [qb[i, pl.ds(16 * l, 16)] * (kb[i, pl.ds(16 * l, 16)] + ev[l])
                  for l in range(8)]
            wb = []
            for h in range(H):
                sh = jnp.sum(pr[2 * h] + pr[2 * h + 1]) * _RS32
                whv = jnp.exp(jnp.full((16,), sh, dtype=jnp.float32))
                wb.append(whv)
                plsc.store_scatter(wqb, [ibc * H + h], whv, mask=lane0)
            for l in range(8):
                msgb[i, pl.ds(16 * l, 16)] = (vb[i, pl.ds(16 * l, 16)] + ev[l]) * wb[l // 2]
        pltpu.sync_copy(msgb, acc_sh.at[dstb], add=True)
        pltpu.sync_copy(wqb, w_hbm.at[pl.ds(off * H, CHUNK * H)])
        return 0

    lax.fori_loop(0, NCHUNK, chunk_body, 0)
    plsc.subcore_barrier()
    pltpu.sync_copy(acc_sh.at[pl.ds(rbase, ROWS_PER_TILE)],
                    part_hbm.at[c, pl.ds(rbase, ROWS_PER_TILE)])


def _edge_pass(q, k, v, src, dst, a0, a1, wet, zeros):
    mesh = plsc.VectorSubcoreMesh(core_axis_name="c", subcore_axis_name="s")
    f = pl.kernel(
        _edge_body,
        mesh=mesh,
        compiler_params=_SC_PARAMS,
        out_type=(
            jax.ShapeDtypeStruct((NC, NPAD, D), jnp.float32),
            jax.ShapeDtypeStruct((H * E,), jnp.float32),
        ),
        scratch_types=[
            pltpu.VMEM_SHARED((NPAD, D), jnp.float32),
            pltpu.VMEM((CHUNK, D), jnp.float32),
            pltpu.VMEM((CHUNK, D), jnp.float32),
            pltpu.VMEM((CHUNK, D), jnp.float32),
            pltpu.VMEM((CHUNK, D), jnp.float32),
            pltpu.VMEM((H * CHUNK,), jnp.float32),
            pltpu.VMEM((CHUNK,), jnp.int32),
            pltpu.VMEM((CHUNK,), jnp.int32),
            pltpu.VMEM((CHUNK,), jnp.float32),
            pltpu.VMEM((CHUNK,), jnp.float32),
            pltpu.VMEM((2, D), jnp.float32),
            pltpu.SemaphoreType.DMA,
        ],
    )
    return f(q, k, v, src, dst, a0, a1, wet, zeros)


def _ssum_body(w_hbm, dst_hbm, zero_hbm, out_hbm, ssumb, wcb, dstb):
    c = lax.axis_index("c")
    s = lax.axis_index("s")
    wid = c * NS + s
    ebase = wid * E_PER_W

    pltpu.sync_copy(zero_hbm.at[pl.ds(0, SSROWS)], ssumb)
    lane = lax.iota(jnp.int32, 16)
    lane4 = lane & 3
    m4 = lane < 4

    def chunk_body(j, _):
        off = ebase + j * CHUNK
        pltpu.sync_copy(dst_hbm.at[pl.ds(off, CHUNK)], dstb)
        pltpu.sync_copy(w_hbm.at[pl.ds(off * H, CHUNK * H)], wcb)

        @plsc.parallel_loop(0, CHUNK, unroll=4)
        def edge_body(i):
            ibc = jnp.full((16,), i, dtype=jnp.int32)
            d = plsc.load_gather(dstb, [ibc])
            wrep = plsc.load_gather(wcb, [ibc * H + lane4])
            flat = d * H + lane4
            plsc.addupdate_scatter(ssumb, [flat >> 7, flat & 127], wrep, mask=m4)

        return 0

    lax.fori_loop(0, NCHUNK, chunk_body, 0)
    pltpu.sync_copy(ssumb, out_hbm.at[wid])


def _ssum_pass(w, dst, zeros):
    mesh = plsc.VectorSubcoreMesh(core_axis_name="c", subcore_axis_name="s")
    f = pl.kernel(
        _ssum_body,
        mesh=mesh,
        compiler_params=_SC_PARAMS,
        out_type=jax.ShapeDtypeStruct((NW, SSROWS, D), jnp.float32),
        scratch_types=[
            pltpu.VMEM((SSROWS, D), jnp.float32),
            pltpu.VMEM((H * CHUNK,), jnp.float32),
            pltpu.VMEM((CHUNK,), jnp.int32),
        ],
    )
    return f(w, dst, zeros)


def _combine_body(p_ref, p2_ref, s_ref, out_ref, stats_ref, acc2):
    i = pl.program_id(0)
    p = p_ref[...]
    accs = p[0] + p[1]
    ws = jnp.sum(p2_ref[...], axis=0)
    blk = ws.shape[0]
    denom = jnp.reshape(jnp.broadcast_to(ws[:, :, None], (blk, H, C)), (blk, D))
    op = accs / (denom + 1e-16) + s_ref[...]
    out_ref[...] = op

    @pl.when(i == 0)
    def _():
        acc2[...] = jnp.zeros_like(acc2)

    acc2[0, :] += jnp.sum(op, axis=0)
    acc2[1, :] += jnp.sum(op * op, axis=0)

    @pl.when(i == pl.num_programs(0) - 1)
    def _():
        stats_ref[...] = acc2[...]


def _combine(parts, parts2, sproj):
    blk = 400
    grid = N // blk
    return pl.pallas_call(
        _combine_body,
        grid=(grid,),
        in_specs=[
            pl.BlockSpec((NC, blk, D), lambda i: (0, i, 0)),
            pl.BlockSpec((NW, blk, H), lambda i: (0, i, 0)),
            pl.BlockSpec((blk, D), lambda i: (i, 0)),
        ],
        out_specs=[
            pl.BlockSpec((blk, D), lambda i: (i, 0)),
            pl.BlockSpec((2, D), lambda i: (0, 0)),
        ],
        out_shape=[
            jax.ShapeDtypeStruct((N, D), jnp.float32),
            jax.ShapeDtypeStruct((2, D), jnp.float32),
        ],
        scratch_shapes=[pltpu.VMEM((2, D), jnp.float32)],
    )(parts, parts2, sproj)


def _bn_body(op_ref, stats_ref, g_ref, b_ref, y_ref):
    st = stats_ref[...]
    mu = st[0:1, :] * (1.0 / N)
    var = st[1:2, :] * (1.0 / N) - mu * mu
    inv = lax.rsqrt(var + 1e-5)
    y = (op_ref[...] - mu) * (inv * g_ref[...]) + b_ref[...]
    y_ref[...] = jnp.where(y > 0, y, 0.01 * y)


def _batchnorm(op, stats, gamma, beta):
    blk = 400
    grid = N // blk
    return pl.pallas_call(
        _bn_body,
        grid=(grid,),
        in_specs=[
            pl.BlockSpec((blk, D), lambda i: (i, 0)),
            pl.BlockSpec((2, D), lambda i: (0, 0)),
            pl.BlockSpec((1, D), lambda i: (0, 0)),
            pl.BlockSpec((1, D), lambda i: (0, 0)),
        ],
        out_specs=pl.BlockSpec((blk, D), lambda i: (i, 0)),
        out_shape=jax.ShapeDtypeStruct((N, D), jnp.float32),
    )(op, stats, gamma, beta)


def kernel(x, edge_idx, edge_attr, Wq, bq, Wk, bk, Wv, bv, We, Ws, bs, gamma, beta):
    src = edge_idx[0].astype(jnp.int32)
    dst = edge_idx[1].astype(jnp.int32)
    bstack = jnp.stack([bq, bk, bv, bs])
    q, k, v, sproj = _projections(x, Wq.T, Wk.T, Wv.T, Ws.T, bstack)
    wet = We.T
    a0 = edge_attr[:, 0]
    a1 = edge_attr[:, 1]
    zeros = jnp.zeros((NPAD, D), jnp.float32)
    parts, w = _edge_pass(q, k, v, src, dst, a0, a1, wet, zeros)
    parts2 = _ssum_pass(w, dst, zeros).reshape(NW, NPAD, H)
    op, stats = _combine(parts[:, :N], parts2[:, :N], sproj)
    return _batchnorm(op, stats, gamma.reshape(1, D), beta.reshape(1, D))
